# baseline (speedup 1.0000x reference)
import functools

import jax
import jax.numpy as jnp
from jax import lax
from jax.experimental import pallas as pl
from jax.experimental.pallas import tpu as pltpu
from jax.experimental.pallas import tpu_sc as plsc

B = 4
S = 2048
D = 1024
NW = 32
SPW = S // NW
PGRP = 16
NPG = SPW // PGRP
NBP = B // 2
STEPS = NPG * NBP
CHUNK = 2 * PGRP
NBUF = 3
LANES = 16

_mesh = plsc.VectorSubcoreMesh(core_axis_name="c", subcore_axis_name="s")


@functools.partial(
    pl.kernel,
    out_type=jax.ShapeDtypeStruct((B * S, D), jnp.float32),
    mesh=_mesh,
    scratch_types=[
        pltpu.VMEM((B, SPW), jnp.int32),
        pltpu.VMEM((PGRP, D), jnp.float32),
        *[pltpu.VMEM((CHUNK, D), jnp.float32) for _ in range(NBUF)],
        *[pltpu.SemaphoreType.DMA for _ in range(NBUF)],
        *[pltpu.SemaphoreType.DMA for _ in range(NBUF)],
        pltpu.SemaphoreType.DMA,
        pltpu.SemaphoreType.DMA,
    ],
)
def _emb_kernel(x_hbm, tok_hbm, pos_hbm, out_hbm, idx_v, pos_v, *rest):
    bufs = rest[:NBUF]
    gsems = rest[NBUF:2 * NBUF]
    wsems = rest[2 * NBUF:3 * NBUF]
    psem, isem = rest[3 * NBUF:]

    cid = lax.axis_index("c")
    sid = lax.axis_index("s")
    wid = sid * 2 + cid

    def pos_piece_load(pg):
        return pltpu.async_copy(
            pos_hbm.at[pl.ds(wid * SPW + pg * PGRP, PGRP)], pos_v, psem)

    def gather(t):
        pg, bp = divmod(t, NBP)
        p = t % NBUF
        ds = []
        for k in range(2):
            b = 2 * bp + k
            idx = idx_v.at[b, pl.ds(pg * PGRP, PGRP)]
            ds.append(pltpu.async_copy(
                tok_hbm.at[idx], bufs[p].at[pl.ds(k * PGRP, PGRP)], gsems[p]))
        return ds

    idx_d = [
        pltpu.async_copy(x_hbm.at[b, pl.ds(wid * SPW, SPW)], idx_v.at[b], isem)
        for b in range(B)
    ]
    pd = pos_piece_load(0)
    for d in idx_d:
        d.wait()

    gd = [None] * NBUF
    wd = [None] * NBUF
    gd[0] = gather(0)

    for t in range(STEPS):
        p = t % NBUF
        pg, bp = divmod(t, NBP)
        if t + 1 < STEPS:
            pn = (t + 1) % NBUF
            for d in wd[pn] or ():
                d.wait()
            wd[pn] = None
            gd[pn] = gather(t + 1)
        for d in gd[p]:
            d.wait()
        if bp == 0:
            pd.wait()
        buf = bufs[p]

        @plsc.parallel_loop(0, PGRP)
        def _add(i):
            @plsc.parallel_loop(0, D, step=LANES, unroll=16)
            def _add_row(o):
                sl = pl.ds(o, LANES)
                v = pos_v[i, sl]
                buf[i, sl] = buf[i, sl] + v
                buf[PGRP + i, sl] = buf[PGRP + i, sl] + v

        halves = []
        for k in range(2):
            b = 2 * bp + k
            row_base = b * S + wid * SPW + pg * PGRP
            halves.append(pltpu.async_copy(
                buf.at[pl.ds(k * PGRP, PGRP)],
                out_hbm.at[pl.ds(row_base, PGRP)], wsems[p]))
        wd[p] = halves

        if bp == NBP - 1 and pg + 1 < NPG:
            pd = pos_piece_load(pg + 1)

    for half in wd:
        for d in half or ():
            d.wait()


def kernel(x, token_table, pos_table):
    out = _emb_kernel(x.astype(jnp.int32), token_table, pos_table)
    return out.reshape(B, S, D)


# speedup vs baseline: 1.0275x; 1.0275x over previous
import functools

import jax
import jax.numpy as jnp
from jax import lax
from jax.experimental import pallas as pl
from jax.experimental.pallas import tpu as pltpu
from jax.experimental.pallas import tpu_sc as plsc

B = 4
S = 2048
D = 1024
NW = 32
SPW = S // NW
PGRP = 16
NPG = SPW // PGRP
NBP = B // 2
STEPS = NPG * NBP
CHUNK = 2 * PGRP
NBUF = 3
LANES = 16

_mesh = plsc.VectorSubcoreMesh(core_axis_name="c", subcore_axis_name="s")


@functools.partial(
    pl.kernel,
    out_type=jax.ShapeDtypeStruct((B * S, D), jnp.float32),
    mesh=_mesh,
    scratch_types=[
        pltpu.VMEM((B, SPW), jnp.int32),
        pltpu.VMEM((PGRP, D), jnp.float32),
        *[pltpu.VMEM((CHUNK, D), jnp.float32) for _ in range(NBUF)],
        *[pltpu.SemaphoreType.DMA for _ in range(NBUF)],
        *[pltpu.SemaphoreType.DMA for _ in range(NBUF)],
        pltpu.SemaphoreType.DMA,
        pltpu.SemaphoreType.DMA,
    ],
)
def _emb_kernel(x_hbm, tok_hbm, pos_hbm, out_hbm, idx_v, pos_v, *rest):
    bufs = rest[:NBUF]
    gsems = rest[NBUF:2 * NBUF]
    wsems = rest[2 * NBUF:3 * NBUF]
    psem, isem = rest[3 * NBUF:]

    cid = lax.axis_index("c")
    sid = lax.axis_index("s")
    wid = sid * 2 + cid

    def pos_piece_load(pg):
        return pltpu.async_copy(
            pos_hbm.at[pl.ds(wid * SPW + pg * PGRP, PGRP)], pos_v, psem)

    def gather(t):
        pg, bp = divmod(t, NBP)
        p = t % NBUF
        ds = []
        for k in range(2):
            b = 2 * bp + k
            idx = idx_v.at[b, pl.ds(pg * PGRP, PGRP)]
            ds.append(pltpu.async_copy(
                tok_hbm.at[idx], bufs[p].at[pl.ds(k * PGRP, PGRP)], gsems[p]))
        return ds

    idx_d = [
        pltpu.async_copy(x_hbm.at[b, pl.ds(wid * SPW, SPW)], idx_v.at[b], isem)
        for b in range(B)
    ]
    pd = pos_piece_load(0)
    for d in idx_d:
        d.wait()

    gd = [None] * NBUF
    wd = [None] * NBUF
    gd[0] = gather(0)

    for t in range(STEPS):
        p = t % NBUF
        pg, bp = divmod(t, NBP)
        if t + 1 < STEPS:
            pn = (t + 1) % NBUF
            for d in wd[pn] or ():
                d.wait()
            wd[pn] = None
            gd[pn] = gather(t + 1)
        for d in gd[p]:
            d.wait()
        if bp == 0:
            pd.wait()
        buf = bufs[p]

        @plsc.parallel_loop(0, PGRP)
        def _add(i):
            @plsc.parallel_loop(0, D, step=LANES, unroll=4)
            def _add_row(o):
                sl = pl.ds(o, LANES)
                v = pos_v[i, sl]
                buf[i, sl] = buf[i, sl] + v
                buf[PGRP + i, sl] = buf[PGRP + i, sl] + v

        halves = []
        for k in range(2):
            b = 2 * bp + k
            row_base = b * S + wid * SPW + pg * PGRP
            halves.append(pltpu.async_copy(
                buf.at[pl.ds(k * PGRP, PGRP)],
                out_hbm.at[pl.ds(row_base, PGRP)], wsems[p]))
        wd[p] = halves

        if bp == NBP - 1 and pg + 1 < NPG:
            pd = pos_piece_load(pg + 1)

    for half in wd:
        for d in half or ():
            d.wait()


def kernel(x, token_table, pos_table):
    out = _emb_kernel(x.astype(jnp.int32), token_table, pos_table)
    return out.reshape(B, S, D)
